# baseline (speedup 1.0000x reference)
import functools

import jax
import jax.numpy as jnp
from jax import lax
from jax.experimental import pallas as pl
from jax.experimental.pallas import tpu as pltpu
from jax.experimental.pallas import tpu_sc as plsc

N_NODES = 10000
E = 320000
N_G = 128
N_X = 16
N_H = 128
N_U = 16
NB = 8
D1 = 144

NC = 2
NS = 16
NW = NC * NS

WIN = 400
NWIN = E // WIN
EG = E // 128


def _f32(*shape):
    return jax.ShapeDtypeStruct(shape, jnp.float32)


def _gather_body(table_hbm, idx_hbm, out_hbm, idx0, idx1, rows0, rows1,
                 isem, gsem):
    wid = lax.axis_index("s") * NC + lax.axis_index("c")
    w_lo = (NWIN * wid) // NW
    w_hi = (NWIN * (wid + 1)) // NW
    npair = (w_hi - w_lo) // 2

    def load_idx(w, idx_v):
        pltpu.async_copy(idx_hbm.at[pl.ds(w * WIN, WIN)], idx_v, isem)

    def drain_idx(idx_v):
        pltpu.make_async_copy(idx_hbm.at[pl.ds(0, WIN)], idx_v, isem).wait()

    @pl.when(w_lo < w_hi)
    def _():
        load_idx(w_lo, idx0)

    def pair_body(p, carry):
        w0 = w_lo + 2 * p
        load_idx(w0 + 1, idx1)
        drain_idx(idx0)
        pltpu.async_copy(table_hbm.at[idx0], rows0, gsem).wait()
        pltpu.sync_copy(rows0, out_hbm.at[pl.ds(w0 * WIN, WIN)])

        @pl.when(w0 + 2 < w_hi)
        def _():
            load_idx(w0 + 2, idx0)

        drain_idx(idx1)
        pltpu.async_copy(table_hbm.at[idx1], rows1, gsem).wait()
        pltpu.sync_copy(rows1, out_hbm.at[pl.ds((w0 + 1) * WIN, WIN)])
        return carry

    lax.fori_loop(0, npair, pair_body, 0)
    wt = w_lo + 2 * npair

    @pl.when(wt < w_hi)
    def _():
        drain_idx(idx0)
        pltpu.async_copy(table_hbm.at[idx0], rows0, gsem).wait()
        pltpu.sync_copy(rows0, out_hbm.at[pl.ds(wt * WIN, WIN)])


def _gather_rows(table, idx):
    mesh = plsc.VectorSubcoreMesh(core_axis_name="c", subcore_axis_name="s")
    k = functools.partial(
        pl.kernel,
        mesh=mesh,
        out_type=_f32(E, N_G),
        scratch_types=[
            pltpu.VMEM((WIN,), jnp.int32),
            pltpu.VMEM((WIN,), jnp.int32),
            pltpu.VMEM((WIN, N_G), jnp.float32),
            pltpu.VMEM((WIN, N_G), jnp.float32),
            pltpu.SemaphoreType.DMA,
            pltpu.SemaphoreType.DMA,
        ],
    )(_gather_body)
    return k(table, idx)


def _edge_body(gt_ref, at_ref, wag_ref, wae_ref, b1a_ref, w1b_ref, b1b_ref,
               u_ref):
    z = (
        jnp.dot(gt_ref[...], wag_ref[...], preferred_element_type=jnp.float32)
        + jnp.dot(at_ref[...], wae_ref[...], preferred_element_type=jnp.float32)
        + b1a_ref[...]
    )
    h = jnp.where(z >= 0, z, 0.1 * z)
    out = jnp.dot(h, w1b_ref[...], preferred_element_type=jnp.float32) + b1b_ref[...]
    o2 = out * out
    o3 = o2 * out
    o4 = o2 * o2
    be = out.shape[0]
    u_ref[0] = out[:, :128]
    u_ref[1] = o2[:, :128]
    u_ref[2] = o3[:, :128]
    u_ref[3] = o4[:, :128]
    u_ref[4] = jnp.concatenate(
        [out[:, 128:], o2[:, 128:], o3[:, 128:], o4[:, 128:],
         jnp.ones((be, 1), jnp.float32),
         jnp.zeros((be, 63), jnp.float32)], axis=1)


def _edge_mlp(gt, edge_attr, w1a_g, w1a_e, b1a_row, w1b, b1b_row):
    be = 2000
    return pl.pallas_call(
        _edge_body,
        grid=(E // be,),
        in_specs=[
            pl.BlockSpec((be, N_G), lambda i: (i, 0)),
            pl.BlockSpec((be, N_X), lambda i: (i, 0)),
            pl.BlockSpec((N_G, D1), lambda i: (0, 0)),
            pl.BlockSpec((N_X, D1), lambda i: (0, 0)),
            pl.BlockSpec((1, D1), lambda i: (0, 0)),
            pl.BlockSpec((D1, D1), lambda i: (0, 0)),
            pl.BlockSpec((1, D1), lambda i: (0, 0)),
        ],
        out_specs=pl.BlockSpec((5, be, 128), lambda i: (0, i, 0)),
        out_shape=_f32(5, E, 128),
    )(gt, edge_attr, w1a_g, w1a_e, b1a_row, w1b, b1b_row)


SCW = 128
NSW = E // SCW
STR = 64
NSTR = 157


def _scatter_body(uflat_hbm, src_hbm, zrows_hbm, sacc_hbm, tail_hbm,
                  buf0, buf1, idx0, idx1, zb, cbuf, acc_sh, lsem, ssem0,
                  ssem1):
    c = lax.axis_index("c")
    s = lax.axis_index("s")
    st_lo = (NSTR * s) // NS
    st_hi = (NSTR * (s + 1)) // NS

    pltpu.sync_copy(zrows_hbm, zb)

    def striped(f):
        def body(st, carry):
            @pl.when(st < NSTR - 1)
            def _():
                f(st * STR, STR)

            @pl.when(st == NSTR - 1)
            def _():
                f((NSTR - 1) * STR, N_NODES - (NSTR - 1) * STR)
            return carry
        lax.fori_loop(st_lo, st_hi, body, 0)

    def zero_acc():
        striped(lambda r0, sz: pltpu.sync_copy(zb.at[pl.ds(0, sz)],
                                               acc_sh.at[pl.ds(r0, sz)]))

    def copy_out(dest_base):
        def f(r0, sz):
            pltpu.sync_copy(acc_sh.at[pl.ds(r0, sz)], cbuf.at[pl.ds(0, sz)])
            pltpu.sync_copy(cbuf.at[pl.ds(0, sz)],
                            sacc_tail_dst.at[pl.ds(dest_base + r0, sz)])
        striped(f)

    def scan_windows(w_lo, w_hi, kbase):
        npair = (w_hi - w_lo) // 2

        def start_load(w, idx_v, buf_v):
            off = w * SCW
            pltpu.async_copy(src_hbm.at[pl.ds(off, SCW)], idx_v, lsem)
            pltpu.async_copy(uflat_hbm.at[pl.ds(kbase + off, SCW)], buf_v,
                             lsem)

        def drain(idx_v, buf_v):
            pltpu.make_async_copy(src_hbm.at[pl.ds(0, SCW)], idx_v,
                                  lsem).wait()
            pltpu.make_async_copy(uflat_hbm.at[pl.ds(0, SCW)], buf_v,
                                  lsem).wait()

        def scat_start(idx_v, buf_v, ssem):
            pltpu.async_copy(buf_v, acc_sh.at[idx_v], ssem, add=True)

        def scat_drain(idx_v, buf_v, ssem):
            pltpu.make_async_copy(buf_v, acc_sh.at[idx_v], ssem).wait()

        @pl.when(w_lo < w_hi)
        def _():
            start_load(w_lo, idx0, buf0)

        def pair_body(p, carry):
            w0 = w_lo + 2 * p
            drain(idx0, buf0)
            scat_start(idx0, buf0, ssem0)

            @pl.when(p > 0)
            def _():
                scat_drain(idx1, buf1, ssem1)

            start_load(w0 + 1, idx1, buf1)
            drain(idx1, buf1)
            scat_start(idx1, buf1, ssem1)
            scat_drain(idx0, buf0, ssem0)

            @pl.when(w0 + 2 < w_hi)
            def _():
                start_load(w0 + 2, idx0, buf0)

            return carry

        lax.fori_loop(0, npair, pair_body, 0)
        wt = w_lo + 2 * npair

        @pl.when(npair > 0)
        def _():
            scat_drain(idx1, buf1, ssem1)

        @pl.when(wt < w_hi)
        def _():
            drain(idx0, buf0)
            scat_start(idx0, buf0, ssem0)
            scat_drain(idx0, buf0, ssem0)

    for r in range(3):
        zero_acc()
        plsc.subcore_barrier()
        if r < 2:
            k = 2 * r + c
            w_lo = (NSW * s) // NS
            w_hi = (NSW * (s + 1)) // NS
            scan_windows(w_lo, w_hi, k * E)
            plsc.subcore_barrier()
            sacc_tail_dst = sacc_hbm
            copy_out(k * N_NODES)
        else:
            hw = NSW // 2
            w_lo = c * hw + (hw * s) // NS
            w_hi = c * hw + (hw * (s + 1)) // NS
            scan_windows(w_lo, w_hi, 4 * E)
            plsc.subcore_barrier()
            sacc_tail_dst = tail_hbm
            copy_out(c * N_NODES)
        plsc.subcore_barrier()


def _segment_moments(uflat, src, zrows):
    mesh = plsc.VectorSubcoreMesh(core_axis_name="c", subcore_axis_name="s")
    k = functools.partial(
        pl.kernel,
        mesh=mesh,
        out_type=(_f32(4 * N_NODES, 128), _f32(2 * N_NODES, 128)),
        scratch_types=[
            pltpu.VMEM((SCW, 128), jnp.float32),
            pltpu.VMEM((SCW, 128), jnp.float32),
            pltpu.VMEM((SCW,), jnp.int32),
            pltpu.VMEM((SCW,), jnp.int32),
            pltpu.VMEM((STR, 128), jnp.float32),
            pltpu.VMEM((STR, 128), jnp.float32),
            pltpu.VMEM_SHARED((N_NODES, 128), jnp.float32),
            pltpu.SemaphoreType.DMA,
            pltpu.SemaphoreType.DMA,
            pltpu.SemaphoreType.DMA,
        ],
    )(_scatter_body)
    return k(uflat, src, zrows)


def _node_body(xh_ref, sacc_ref, ta_ref, tb_ref, bh_ref, u_ref,
               w2h_ref, w2n_ref, w2a_ref, w2b_ref, w2c_ref, w2d_ref, w2u_ref,
               b2a_ref, w2b2_ref, b2b_ref, o_ref):
    t = ta_ref[...] + tb_ref[...]
    s1 = jnp.concatenate([sacc_ref[0], t[:, 0:16]], axis=1)
    s2 = jnp.concatenate([sacc_ref[1], t[:, 16:32]], axis=1)
    s3 = jnp.concatenate([sacc_ref[2], t[:, 32:48]], axis=1)
    s4 = jnp.concatenate([sacc_ref[3], t[:, 48:64]], axis=1)
    n = t[:, 64:65]
    rden = 1.0 / jnp.maximum(n, 1.0)
    a = s1 * rden
    m2 = s2 * rden
    b2 = 1e-6 + jnp.maximum(m2 - a * a, 0.0)
    b = jnp.sqrt(b2)
    a2 = a * a
    cm3 = s3 - 3.0 * a * s2 + 3.0 * a2 * s1 - n * (a2 * a)
    cm4 = s4 - 4.0 * a * s3 + 6.0 * a2 * s2 - 4.0 * (a2 * a) * s1 + n * (a2 * a2)
    cc = cm3 * rden / (b * b2)
    dd = cm4 * rden / (b2 * b2)
    onehot = (bh_ref[...] == lax.broadcasted_iota(jnp.int32, (bh_ref.shape[0], NB), 1)
              ).astype(jnp.float32)
    ug = jnp.dot(onehot, u_ref[...], preferred_element_type=jnp.float32)
    f = (
        jnp.dot(xh_ref[...], w2h_ref[...], preferred_element_type=jnp.float32)
        + n * w2n_ref[...]
        + jnp.dot(a, w2a_ref[...], preferred_element_type=jnp.float32)
        + jnp.dot(b, w2b_ref[...], preferred_element_type=jnp.float32)
        + jnp.dot(cc, w2c_ref[...], preferred_element_type=jnp.float32)
        + jnp.dot(dd, w2d_ref[...], preferred_element_type=jnp.float32)
        + jnp.dot(ug, w2u_ref[...], preferred_element_type=jnp.float32)
        + b2a_ref[...]
    )
    f = jnp.where(f >= 0, f, 0.1 * f)
    o_ref[...] = (
        jnp.dot(f, w2b2_ref[...], preferred_element_type=jnp.float32) + b2b_ref[...]
    )


def _node_mlp(x_h, sacc, ta, tb, bh2, u, w2_parts, b2a_row, w2b, b2b_row):
    bn = 1000
    w2h, w2n, w2a, w2bp, w2c, w2d, w2u = w2_parts
    full = lambda shape: pl.BlockSpec(shape, lambda i: tuple(0 for _ in shape))
    return pl.pallas_call(
        _node_body,
        grid=(N_NODES // bn,),
        in_specs=[
            pl.BlockSpec((bn, N_H), lambda i: (i, 0)),
            pl.BlockSpec((4, bn, 128), lambda i: (0, i, 0)),
            pl.BlockSpec((bn, 128), lambda i: (i, 0)),
            pl.BlockSpec((bn, 128), lambda i: (i, 0)),
            pl.BlockSpec((bn, 1), lambda i: (i, 0)),
            full((NB, N_U)),
            full((N_H, N_H)),
            full((1, N_H)),
            full((D1, N_H)),
            full((D1, N_H)),
            full((D1, N_H)),
            full((D1, N_H)),
            full((N_U, N_H)),
            full((1, N_H)),
            full((N_H, N_H)),
            full((1, N_H)),
        ],
        out_specs=pl.BlockSpec((bn, N_H), lambda i: (i, 0)),
        out_shape=_f32(N_NODES, N_H),
    )(x_h, sacc, ta, tb, bh2, u, w2h, w2n, w2a, w2bp, w2c, w2d, w2u,
      b2a_row, w2b, b2b_row)


def kernel(x_h, x_g, edge_index, edge_attr, u, batch_h,
           W1a, b1a, W1b, b1b, W2a, b2a, W2b, b2b):
    src = edge_index[0]
    tgt = edge_index[1]
    b1a_row = b1a.reshape(1, D1)
    b1b_row = b1b.reshape(1, D1)
    b2a_row = b2a.reshape(1, N_H)
    b2b_row = b2b.reshape(1, N_H)

    gt = _gather_rows(x_g, tgt)
    uarr = _edge_mlp(gt, edge_attr, W1a[:N_G], W1a[N_G:],
                     b1a_row, W1b, b1b_row)

    zrows = jnp.zeros((STR, 128), jnp.float32)
    sacc, tail = _segment_moments(uarr.reshape(5 * E, 128), src, zrows)

    sacc = sacc.reshape(4, N_NODES, 128)
    ta = tail[:N_NODES]
    tb = tail[N_NODES:]
    bh2 = batch_h.reshape(N_NODES, 1)
    w2_parts = (W2a[:N_H], W2a[N_H:N_H + 1], W2a[N_H + 1:N_H + 1 + D1],
                W2a[N_H + 1 + D1:N_H + 1 + 2 * D1],
                W2a[N_H + 1 + 2 * D1:N_H + 1 + 3 * D1],
                W2a[N_H + 1 + 3 * D1:N_H + 1 + 4 * D1],
                W2a[N_H + 1 + 4 * D1:])
    return _node_mlp(x_h, sacc, ta, tb, bh2, u, w2_parts, b2a_row, W2b, b2b_row)


# speedup vs baseline: 1.0502x; 1.0502x over previous
import functools

import jax
import jax.numpy as jnp
from jax import lax
from jax.experimental import pallas as pl
from jax.experimental.pallas import tpu as pltpu
from jax.experimental.pallas import tpu_sc as plsc

N_NODES = 10000
E = 320000
N_G = 128
N_X = 16
N_H = 128
N_U = 16
NB = 8
D1 = 144

NC = 2
NS = 16
NW = NC * NS

WIN = 400
NWIN = E // WIN
EG = E // 128


def _f32(*shape):
    return jax.ShapeDtypeStruct(shape, jnp.float32)


def _gather_body(table_hbm, idx_hbm, out_hbm, idx0, idx1, rows0, rows1,
                 isem, gsem):
    wid = lax.axis_index("s") * NC + lax.axis_index("c")
    w_lo = (NWIN * wid) // NW
    w_hi = (NWIN * (wid + 1)) // NW
    npair = (w_hi - w_lo) // 2

    def load_idx(w, idx_v):
        pltpu.async_copy(idx_hbm.at[pl.ds(w * WIN, WIN)], idx_v, isem)

    def drain_idx(idx_v):
        pltpu.make_async_copy(idx_hbm.at[pl.ds(0, WIN)], idx_v, isem).wait()

    @pl.when(w_lo < w_hi)
    def _():
        load_idx(w_lo, idx0)

    def pair_body(p, carry):
        w0 = w_lo + 2 * p
        load_idx(w0 + 1, idx1)
        drain_idx(idx0)
        pltpu.async_copy(table_hbm.at[idx0], rows0, gsem).wait()
        pltpu.sync_copy(rows0, out_hbm.at[pl.ds(w0 * WIN, WIN)])

        @pl.when(w0 + 2 < w_hi)
        def _():
            load_idx(w0 + 2, idx0)

        drain_idx(idx1)
        pltpu.async_copy(table_hbm.at[idx1], rows1, gsem).wait()
        pltpu.sync_copy(rows1, out_hbm.at[pl.ds((w0 + 1) * WIN, WIN)])
        return carry

    lax.fori_loop(0, npair, pair_body, 0)
    wt = w_lo + 2 * npair

    @pl.when(wt < w_hi)
    def _():
        drain_idx(idx0)
        pltpu.async_copy(table_hbm.at[idx0], rows0, gsem).wait()
        pltpu.sync_copy(rows0, out_hbm.at[pl.ds(wt * WIN, WIN)])


def _gather_rows(table, idx):
    mesh = plsc.VectorSubcoreMesh(core_axis_name="c", subcore_axis_name="s")
    k = functools.partial(
        pl.kernel,
        mesh=mesh,
        out_type=_f32(E, N_G),
        scratch_types=[
            pltpu.VMEM((WIN,), jnp.int32),
            pltpu.VMEM((WIN,), jnp.int32),
            pltpu.VMEM((WIN, N_G), jnp.float32),
            pltpu.VMEM((WIN, N_G), jnp.float32),
            pltpu.SemaphoreType.DMA,
            pltpu.SemaphoreType.DMA,
        ],
    )(_gather_body)
    return k(table, idx)


def _edge_body(gt_ref, at_ref, wag_ref, wae_ref, b1a_ref, w1b_ref, b1b_ref,
               u_ref):
    z = (
        jnp.dot(gt_ref[...], wag_ref[...], preferred_element_type=jnp.float32)
        + jnp.dot(at_ref[...], wae_ref[...], preferred_element_type=jnp.float32)
        + b1a_ref[...]
    )
    h = jnp.where(z >= 0, z, 0.1 * z)
    out = jnp.dot(h, w1b_ref[...], preferred_element_type=jnp.float32) + b1b_ref[...]
    o2 = out * out
    o3 = o2 * out
    o4 = o2 * o2
    be = out.shape[0]
    u_ref[0] = out[:, :128]
    u_ref[1] = o2[:, :128]
    u_ref[2] = o3[:, :128]
    u_ref[3] = o4[:, :128]
    u_ref[4] = jnp.concatenate(
        [out[:, 128:], o2[:, 128:], o3[:, 128:], o4[:, 128:],
         jnp.ones((be, 1), jnp.float32),
         jnp.zeros((be, 63), jnp.float32)], axis=1)


def _edge_mlp(gt, edge_attr, w1a_g, w1a_e, b1a_row, w1b, b1b_row):
    be = 4000
    return pl.pallas_call(
        _edge_body,
        grid=(E // be,),
        in_specs=[
            pl.BlockSpec((be, N_G), lambda i: (i, 0)),
            pl.BlockSpec((be, N_X), lambda i: (i, 0)),
            pl.BlockSpec((N_G, D1), lambda i: (0, 0)),
            pl.BlockSpec((N_X, D1), lambda i: (0, 0)),
            pl.BlockSpec((1, D1), lambda i: (0, 0)),
            pl.BlockSpec((D1, D1), lambda i: (0, 0)),
            pl.BlockSpec((1, D1), lambda i: (0, 0)),
        ],
        out_specs=pl.BlockSpec((5, be, 128), lambda i: (0, i, 0)),
        out_shape=_f32(5, E, 128),
    )(gt, edge_attr, w1a_g, w1a_e, b1a_row, w1b, b1b_row)


SCW = 128
NSW = E // SCW
STR = 64
NSTR = 157


def _scatter_body(uflat_hbm, src_hbm, zrows_hbm, sacc_hbm, tail_hbm,
                  buf0, buf1, idx0, idx1, zb, cbuf, acc_sh, lsem, ssem0,
                  ssem1):
    c = lax.axis_index("c")
    s = lax.axis_index("s")
    st_lo = (NSTR * s) // NS
    st_hi = (NSTR * (s + 1)) // NS

    pltpu.sync_copy(zrows_hbm, zb)

    def striped(f):
        def body(st, carry):
            @pl.when(st < NSTR - 1)
            def _():
                f(st * STR, STR)

            @pl.when(st == NSTR - 1)
            def _():
                f((NSTR - 1) * STR, N_NODES - (NSTR - 1) * STR)
            return carry
        lax.fori_loop(st_lo, st_hi, body, 0)

    def zero_acc():
        striped(lambda r0, sz: pltpu.sync_copy(zb.at[pl.ds(0, sz)],
                                               acc_sh.at[pl.ds(r0, sz)]))

    def copy_out(dest_base):
        def f(r0, sz):
            pltpu.sync_copy(acc_sh.at[pl.ds(r0, sz)], cbuf.at[pl.ds(0, sz)])
            pltpu.sync_copy(cbuf.at[pl.ds(0, sz)],
                            sacc_tail_dst.at[pl.ds(dest_base + r0, sz)])
        striped(f)

    def scan_windows(w_lo, w_hi, kbase):
        npair = (w_hi - w_lo) // 2

        def start_load(w, idx_v, buf_v):
            off = w * SCW
            pltpu.async_copy(src_hbm.at[pl.ds(off, SCW)], idx_v, lsem)
            pltpu.async_copy(uflat_hbm.at[pl.ds(kbase + off, SCW)], buf_v,
                             lsem)

        def drain(idx_v, buf_v):
            pltpu.make_async_copy(src_hbm.at[pl.ds(0, SCW)], idx_v,
                                  lsem).wait()
            pltpu.make_async_copy(uflat_hbm.at[pl.ds(0, SCW)], buf_v,
                                  lsem).wait()

        def scat_start(idx_v, buf_v, ssem):
            pltpu.async_copy(buf_v, acc_sh.at[idx_v], ssem, add=True)

        def scat_drain(idx_v, buf_v, ssem):
            pltpu.make_async_copy(buf_v, acc_sh.at[idx_v], ssem).wait()

        @pl.when(w_lo < w_hi)
        def _():
            start_load(w_lo, idx0, buf0)

        def pair_body(p, carry):
            w0 = w_lo + 2 * p
            drain(idx0, buf0)
            scat_start(idx0, buf0, ssem0)

            @pl.when(p > 0)
            def _():
                scat_drain(idx1, buf1, ssem1)

            start_load(w0 + 1, idx1, buf1)
            drain(idx1, buf1)
            scat_start(idx1, buf1, ssem1)
            scat_drain(idx0, buf0, ssem0)

            @pl.when(w0 + 2 < w_hi)
            def _():
                start_load(w0 + 2, idx0, buf0)

            return carry

        lax.fori_loop(0, npair, pair_body, 0)
        wt = w_lo + 2 * npair

        @pl.when(npair > 0)
        def _():
            scat_drain(idx1, buf1, ssem1)

        @pl.when(wt < w_hi)
        def _():
            drain(idx0, buf0)
            scat_start(idx0, buf0, ssem0)
            scat_drain(idx0, buf0, ssem0)

    for r in range(3):
        zero_acc()
        plsc.subcore_barrier()
        if r < 2:
            k = 2 * r + c
            w_lo = (NSW * s) // NS
            w_hi = (NSW * (s + 1)) // NS
            scan_windows(w_lo, w_hi, k * E)
            plsc.subcore_barrier()
            sacc_tail_dst = sacc_hbm
            copy_out(k * N_NODES)
        else:
            hw = NSW // 2
            w_lo = c * hw + (hw * s) // NS
            w_hi = c * hw + (hw * (s + 1)) // NS
            scan_windows(w_lo, w_hi, 4 * E)
            plsc.subcore_barrier()
            sacc_tail_dst = tail_hbm
            copy_out(c * N_NODES)
        plsc.subcore_barrier()


def _segment_moments(uflat, src, zrows):
    mesh = plsc.VectorSubcoreMesh(core_axis_name="c", subcore_axis_name="s")
    k = functools.partial(
        pl.kernel,
        mesh=mesh,
        out_type=(_f32(4 * N_NODES, 128), _f32(2 * N_NODES, 128)),
        scratch_types=[
            pltpu.VMEM((SCW, 128), jnp.float32),
            pltpu.VMEM((SCW, 128), jnp.float32),
            pltpu.VMEM((SCW,), jnp.int32),
            pltpu.VMEM((SCW,), jnp.int32),
            pltpu.VMEM((STR, 128), jnp.float32),
            pltpu.VMEM((STR, 128), jnp.float32),
            pltpu.VMEM_SHARED((N_NODES, 128), jnp.float32),
            pltpu.SemaphoreType.DMA,
            pltpu.SemaphoreType.DMA,
            pltpu.SemaphoreType.DMA,
        ],
    )(_scatter_body)
    return k(uflat, src, zrows)


def _node_body(xh_ref, sacc_ref, ta_ref, tb_ref, bh_ref, u_ref,
               w2h_ref, w2n_ref, w2a_ref, w2b_ref, w2c_ref, w2d_ref, w2u_ref,
               b2a_ref, w2b2_ref, b2b_ref, o_ref):
    t = ta_ref[...] + tb_ref[...]
    s1 = jnp.concatenate([sacc_ref[0], t[:, 0:16]], axis=1)
    s2 = jnp.concatenate([sacc_ref[1], t[:, 16:32]], axis=1)
    s3 = jnp.concatenate([sacc_ref[2], t[:, 32:48]], axis=1)
    s4 = jnp.concatenate([sacc_ref[3], t[:, 48:64]], axis=1)
    n = t[:, 64:65]
    rden = 1.0 / jnp.maximum(n, 1.0)
    a = s1 * rden
    m2 = s2 * rden
    b2 = 1e-6 + jnp.maximum(m2 - a * a, 0.0)
    b = jnp.sqrt(b2)
    a2 = a * a
    cm3 = s3 - 3.0 * a * s2 + 3.0 * a2 * s1 - n * (a2 * a)
    cm4 = s4 - 4.0 * a * s3 + 6.0 * a2 * s2 - 4.0 * (a2 * a) * s1 + n * (a2 * a2)
    cc = cm3 * rden / (b * b2)
    dd = cm4 * rden / (b2 * b2)
    onehot = (bh_ref[...] == lax.broadcasted_iota(jnp.int32, (bh_ref.shape[0], NB), 1)
              ).astype(jnp.float32)
    ug = jnp.dot(onehot, u_ref[...], preferred_element_type=jnp.float32)
    f = (
        jnp.dot(xh_ref[...], w2h_ref[...], preferred_element_type=jnp.float32)
        + n * w2n_ref[...]
        + jnp.dot(a, w2a_ref[...], preferred_element_type=jnp.float32)
        + jnp.dot(b, w2b_ref[...], preferred_element_type=jnp.float32)
        + jnp.dot(cc, w2c_ref[...], preferred_element_type=jnp.float32)
        + jnp.dot(dd, w2d_ref[...], preferred_element_type=jnp.float32)
        + jnp.dot(ug, w2u_ref[...], preferred_element_type=jnp.float32)
        + b2a_ref[...]
    )
    f = jnp.where(f >= 0, f, 0.1 * f)
    o_ref[...] = (
        jnp.dot(f, w2b2_ref[...], preferred_element_type=jnp.float32) + b2b_ref[...]
    )


def _node_mlp(x_h, sacc, ta, tb, bh2, u, w2_parts, b2a_row, w2b, b2b_row):
    bn = 1000
    w2h, w2n, w2a, w2bp, w2c, w2d, w2u = w2_parts
    full = lambda shape: pl.BlockSpec(shape, lambda i: tuple(0 for _ in shape))
    return pl.pallas_call(
        _node_body,
        grid=(N_NODES // bn,),
        in_specs=[
            pl.BlockSpec((bn, N_H), lambda i: (i, 0)),
            pl.BlockSpec((4, bn, 128), lambda i: (0, i, 0)),
            pl.BlockSpec((bn, 128), lambda i: (i, 0)),
            pl.BlockSpec((bn, 128), lambda i: (i, 0)),
            pl.BlockSpec((bn, 1), lambda i: (i, 0)),
            full((NB, N_U)),
            full((N_H, N_H)),
            full((1, N_H)),
            full((D1, N_H)),
            full((D1, N_H)),
            full((D1, N_H)),
            full((D1, N_H)),
            full((N_U, N_H)),
            full((1, N_H)),
            full((N_H, N_H)),
            full((1, N_H)),
        ],
        out_specs=pl.BlockSpec((bn, N_H), lambda i: (i, 0)),
        out_shape=_f32(N_NODES, N_H),
    )(x_h, sacc, ta, tb, bh2, u, w2h, w2n, w2a, w2bp, w2c, w2d, w2u,
      b2a_row, w2b, b2b_row)


def kernel(x_h, x_g, edge_index, edge_attr, u, batch_h,
           W1a, b1a, W1b, b1b, W2a, b2a, W2b, b2b):
    src = edge_index[0]
    tgt = edge_index[1]
    b1a_row = b1a.reshape(1, D1)
    b1b_row = b1b.reshape(1, D1)
    b2a_row = b2a.reshape(1, N_H)
    b2b_row = b2b.reshape(1, N_H)

    gt = _gather_rows(x_g, tgt)
    uarr = _edge_mlp(gt, edge_attr, W1a[:N_G], W1a[N_G:],
                     b1a_row, W1b, b1b_row)

    zrows = jnp.zeros((STR, 128), jnp.float32)
    sacc, tail = _segment_moments(uarr.reshape(5 * E, 128), src, zrows)

    sacc = sacc.reshape(4, N_NODES, 128)
    ta = tail[:N_NODES]
    tb = tail[N_NODES:]
    bh2 = batch_h.reshape(N_NODES, 1)
    w2_parts = (W2a[:N_H], W2a[N_H:N_H + 1], W2a[N_H + 1:N_H + 1 + D1],
                W2a[N_H + 1 + D1:N_H + 1 + 2 * D1],
                W2a[N_H + 1 + 2 * D1:N_H + 1 + 3 * D1],
                W2a[N_H + 1 + 3 * D1:N_H + 1 + 4 * D1],
                W2a[N_H + 1 + 4 * D1:])
    return _node_mlp(x_h, sacc, ta, tb, bh2, u, w2_parts, b2a_row, W2b, b2b_row)
